# baseline (speedup 1.0000x reference)
import functools

import jax
import jax.numpy as jnp
from jax import lax
from jax.experimental import pallas as pl
from jax.experimental.pallas import tpu as pltpu
from jax.experimental.pallas import tpu_sc as plsc

N = 10000
D = 128
E = 320000
EL = 200000

NC = 2
NS = 16
NW = NC * NS

NR = 10240
STRIPE = NR // NS
EPW = 10240
EPAD = NW * EPW

ELW = 6400
ELPAD = NW * ELW

_MESH = dict(core_axis_name="c", subcore_axis_name="s",
             num_cores=NC, num_subcores=NS)


def _seg_sum_body(with_gather, *refs):
    if with_gather:
        (src_hbm, dst_hbm, x_hbm, z_hbm, agg_out,
         sidx_a, didx_a, rows_a, sidx_b, didx_b, rows_b,
         agg_sp, sem_a, sem_b) = refs
        rows_w = rows_a
    else:
        (dst_hbm, z_hbm, agg_out,
         didx_w, rows_w, agg_sp, sem) = refs

    c = lax.axis_index("c")
    s = lax.axis_index("s")
    w = s * NC + c

    r0 = s * STRIPE
    for q in range(STRIPE // 128):
        pltpu.sync_copy(z_hbm.at[pl.ds(r0 + q * 128, 128)], rows_w)
        pltpu.sync_copy(rows_w, agg_sp.at[pl.ds(r0 + q * 128, 128)])
    plsc.subcore_barrier()

    base = w * EPW
    if not with_gather:
        ones = jnp.ones((16,), jnp.float32)

        def fill(i, carry):
            for j in range(D // 16):
                rows_w[i, pl.ds(j * 16, 16)] = ones
            return carry

        lax.fori_loop(0, 128, fill, 0)

    if with_gather:
        nburst = EPW // 128

        pltpu.sync_copy(src_hbm.at[pl.ds(base, 128)], sidx_a)
        pltpu.sync_copy(dst_hbm.at[pl.ds(base, 128)], didx_a)
        pltpu.async_copy(x_hbm.at[sidx_a], rows_a, sem_a)

        def pair(p, carry):
            offb = base + (p * 2 + 1) * 128
            offa2 = base + lax.rem(p * 2 + 2, nburst) * 128
            pltpu.sync_copy(src_hbm.at[pl.ds(offb, 128)], sidx_b)
            pltpu.sync_copy(dst_hbm.at[pl.ds(offb, 128)], didx_b)
            pltpu.make_async_copy(x_hbm.at[sidx_a], rows_a, sem_a).wait()
            pltpu.async_copy(x_hbm.at[sidx_b], rows_b, sem_b)
            pltpu.sync_copy(rows_a, agg_sp.at[didx_a], add=True)
            pltpu.sync_copy(src_hbm.at[pl.ds(offa2, 128)], sidx_a)
            pltpu.sync_copy(dst_hbm.at[pl.ds(offa2, 128)], didx_a)
            pltpu.make_async_copy(x_hbm.at[sidx_b], rows_b, sem_b).wait()
            pltpu.async_copy(x_hbm.at[sidx_a], rows_a, sem_a)
            pltpu.sync_copy(rows_b, agg_sp.at[didx_b], add=True)
            return carry

        lax.fori_loop(0, nburst // 2, pair, 0)
        pltpu.make_async_copy(x_hbm.at[sidx_a], rows_a, sem_a).wait()
    else:
        def burst(i, carry):
            off = base + i * 128
            pltpu.sync_copy(dst_hbm.at[pl.ds(off, 128)], didx_w)
            pltpu.sync_copy(rows_w, agg_sp.at[didx_w], add=True)
            return carry

        lax.fori_loop(0, EPW // 128, burst, 0)
    plsc.subcore_barrier()

    out_base = c * NR + r0
    for q in range(STRIPE // 128):
        pltpu.sync_copy(agg_sp.at[pl.ds(r0 + q * 128, 128)], rows_w)
        pltpu.sync_copy(rows_w, agg_out.at[pl.ds(out_base + q * 128, 128)])


def _make_seg_sum(with_gather):
    if with_gather:
        scratch = [
            pltpu.VMEM((128,), jnp.int32),
            pltpu.VMEM((128,), jnp.int32),
            pltpu.VMEM((128, D), jnp.float32),
            pltpu.VMEM((128,), jnp.int32),
            pltpu.VMEM((128,), jnp.int32),
            pltpu.VMEM((128, D), jnp.float32),
            pltpu.VMEM_SHARED((NR, D), jnp.float32),
            pltpu.SemaphoreType.DMA,
            pltpu.SemaphoreType.DMA,
        ]
    else:
        scratch = [
            pltpu.VMEM((128,), jnp.int32),
            pltpu.VMEM((128, D), jnp.float32),
            pltpu.VMEM_SHARED((NR, D), jnp.float32),
            pltpu.SemaphoreType.DMA,
        ]
    return pl.kernel(
        functools.partial(_seg_sum_body, with_gather),
        out_type=jax.ShapeDtypeStruct((NC * NR, D), jnp.float32),
        mesh=plsc.VectorSubcoreMesh(**_MESH),
        scratch_types=scratch,
    )


def _cls_body(h_hbm, e0_hbm, e1_hbm, pred_out,
              i0_a, i1_a, a_a, b_a, i0_b, i1_b, a_b, b_b,
              out_v, sem_a, sem_b):
    c = lax.axis_index("c")
    s = lax.axis_index("s")
    w = s * NC + c
    base = w * ELW
    nburst = ELW // 128
    lanes = lax.iota(jnp.int32, 16)

    def fire(i0_w, i1_w, a_v, b_v, sem, off):
        pltpu.sync_copy(e0_hbm.at[pl.ds(off, 128)], i0_w)
        pltpu.sync_copy(e1_hbm.at[pl.ds(off, 128)], i1_w)
        pltpu.async_copy(h_hbm.at[i0_w], a_v, sem)
        pltpu.async_copy(h_hbm.at[i1_w], b_v, sem)

    def drain(i0_w, i1_w, a_v, b_v, sem):
        pltpu.make_async_copy(h_hbm.at[i0_w], a_v, sem).wait()
        pltpu.make_async_copy(h_hbm.at[i1_w], b_v, sem).wait()

    def compute(a_v, b_v, off):
        def grp(g, carry2):
            res = jnp.zeros((16,), jnp.float32)
            for r16 in range(16):
                r = g * 16 + r16
                acc = a_v[r, pl.ds(0, 16)] * b_v[r, pl.ds(0, 16)]
                for j in range(1, D // 16):
                    acc = acc + (a_v[r, pl.ds(j * 16, 16)]
                                 * b_v[r, pl.ds(j * 16, 16)])
                for k in (8, 4, 2, 1):
                    acc = acc + jnp.take(acc, lanes ^ k)
                res = jnp.where(lanes == r16, acc, res)
            out_v[pl.ds(g * 16, 16)] = res
            return carry2

        lax.fori_loop(0, 8, grp, 0)
        pltpu.sync_copy(out_v, pred_out.at[pl.ds(off, 128)])

    fire(i0_a, i1_a, a_a, b_a, sem_a, base)

    def pair(p, carry):
        offa = base + (p * 2) * 128
        offb = base + (p * 2 + 1) * 128
        offa2 = base + lax.rem(p * 2 + 2, nburst) * 128
        drain(i0_a, i1_a, a_a, b_a, sem_a)
        fire(i0_b, i1_b, a_b, b_b, sem_b, offb)
        compute(a_a, b_a, offa)
        drain(i0_b, i1_b, a_b, b_b, sem_b)
        fire(i0_a, i1_a, a_a, b_a, sem_a, offa2)
        compute(a_b, b_b, offb)
        return carry

    lax.fori_loop(0, nburst // 2, pair, 0)
    drain(i0_a, i1_a, a_a, b_a, sem_a)


_cls_kernel = pl.kernel(
    _cls_body,
    out_type=jax.ShapeDtypeStruct((ELPAD,), jnp.float32),
    mesh=plsc.VectorSubcoreMesh(**_MESH),
    scratch_types=[
        pltpu.VMEM((128,), jnp.int32),
        pltpu.VMEM((128,), jnp.int32),
        pltpu.VMEM((128, D), jnp.float32),
        pltpu.VMEM((128, D), jnp.float32),
        pltpu.VMEM((128,), jnp.int32),
        pltpu.VMEM((128,), jnp.int32),
        pltpu.VMEM((128, D), jnp.float32),
        pltpu.VMEM((128, D), jnp.float32),
        pltpu.VMEM((128,), jnp.float32),
        pltpu.SemaphoreType.DMA,
        pltpu.SemaphoreType.DMA,
    ],
)


def _tc_body(relu, agg_ref, cnt_ref, x_ref, wl_ref, wr_ref, bl_ref, out_ref):
    aggs = agg_ref[0] + agg_ref[1]
    cnt = cnt_ref[0, :, 0:1] + cnt_ref[1, :, 0:1]
    mean = aggs / jnp.maximum(cnt, 1.0)
    h = lax.dot_general(mean, wl_ref[...], (((1,), (1,)), ((), ())),
                        preferred_element_type=jnp.float32)
    h = h + bl_ref[...]
    h = h + lax.dot_general(x_ref[...], wr_ref[...], (((1,), (1,)), ((), ())),
                            preferred_element_type=jnp.float32)
    if relu:
        h = jnp.maximum(h, 0.0)
    out_ref[...] = h


def _tc_layer(relu, agg, cnt, x, wl, wr, bl):
    R = 1000
    grid = (N // R,)
    return pl.pallas_call(
        functools.partial(_tc_body, relu),
        grid=grid,
        in_specs=[
            pl.BlockSpec((NC, R, D), lambda i: (0, i, 0)),
            pl.BlockSpec((NC, R, D), lambda i: (0, i, 0)),
            pl.BlockSpec((R, D), lambda i: (i, 0)),
            pl.BlockSpec((D, D), lambda i: (0, 0)),
            pl.BlockSpec((D, D), lambda i: (0, 0)),
            pl.BlockSpec((1, D), lambda i: (0, 0)),
        ],
        out_specs=pl.BlockSpec((R, D), lambda i: (i, 0)),
        out_shape=jax.ShapeDtypeStruct((N, D), jnp.float32),
    )(agg, cnt, x, wl, wr, bl)


_seg_sum = _make_seg_sum(True)
_cnt_sum = _make_seg_sum(False)


def kernel(x, edge_index, edge_label_index, Wl1, bl1, Wr1, Wl2, bl2, Wr2):
    ei = edge_index.astype(jnp.int32)
    eli = edge_label_index.astype(jnp.int32)

    pad = EPAD - E
    src = jnp.concatenate([ei[0], jnp.zeros((pad,), jnp.int32)])
    dst = jnp.concatenate([ei[1], jnp.full((pad,), N, jnp.int32)])

    z128 = jnp.zeros((NR, D), jnp.float32)

    agg1 = _seg_sum(src, dst, x, z128).reshape(NC, NR, D)
    cnt = _cnt_sum(dst, z128).reshape(NC, NR, D)
    h1 = _tc_layer(True, agg1, cnt, x, Wl1, Wr1, bl1.reshape(1, D))
    agg2 = _seg_sum(src, dst, h1, z128).reshape(NC, NR, D)
    h2 = _tc_layer(False, agg2, cnt, h1, Wl2, Wr2, bl2.reshape(1, D))

    lpad = ELPAD - EL
    e0 = jnp.concatenate([eli[0], jnp.zeros((lpad,), jnp.int32)])
    e1 = jnp.concatenate([eli[1], jnp.zeros((lpad,), jnp.int32)])
    pred = _cls_kernel(h2, e0, e1)
    return pred[:EL]
